# baseline (speedup 1.0000x reference)
import functools

import jax
import jax.numpy as jnp
from jax import lax
from jax.experimental import pallas as pl
from jax.experimental.pallas import tpu as pltpu
from jax.experimental.pallas import tpu_sc as plsc

B = 1024
D = 256
N = 100000
TEMP = 0.05
INV_TEMP = 1.0 / TEMP

ROWS_PER_BLOCK = 4000
NUM_BLOCKS = N // ROWS_PER_BLOCK

LOG2E = 1.4426950408889634
EXP2_SCALE = INV_TEMP * LOG2E


def _make_sc_gather():
  info = plsc.get_sparse_core_info()
  nw = info.num_cores * info.num_subcores
  b_per_w = B // nw
  mesh = plsc.VectorSubcoreMesh(core_axis_name="c", subcore_axis_name="s")

  @functools.partial(
      pl.kernel,
      mesh=mesh,
      out_type=[
          jax.ShapeDtypeStruct((B, D), jnp.float32),
          jax.ShapeDtypeStruct((B, D), jnp.float32),
      ],
      scratch_types=[
          pltpu.VMEM((b_per_w,), jnp.int32),
          pltpu.VMEM((b_per_w, D), jnp.float32),
          pltpu.VMEM((b_per_w, D), jnp.float32),
          pltpu.SemaphoreType.DMA,
          pltpu.SemaphoreType.DMA,
      ],
  )
  def sc_gather(pos_hbm, neg_hbm, tgt_hbm, out_p, out_n,
                idx_v, rows_p, rows_n, sem_p, sem_n):
    wid = lax.axis_index("s") * info.num_cores + lax.axis_index("c")
    base = wid * b_per_w
    pltpu.sync_copy(tgt_hbm.at[pl.ds(base, b_per_w)], idx_v)
    dma_p = pltpu.async_copy(pos_hbm.at[idx_v], rows_p, sem_p)
    dma_n = pltpu.async_copy(neg_hbm.at[idx_v], rows_n, sem_n)
    dma_p.wait()
    dma_n.wait()
    pltpu.sync_copy(rows_p, out_p.at[pl.ds(base, b_per_w)])
    pltpu.sync_copy(rows_n, out_n.at[pl.ds(base, b_per_w)])

  return sc_gather


_sc_gather_cache = []


def _sc_gather(pos, neg, tgt):
  if not _sc_gather_cache:
    _sc_gather_cache.append(_make_sc_gather())
  return _sc_gather_cache[0](pos, neg, tgt)


def _tc_body(x_ref, neg_ref, cp_ref, cn_ref, out_ref,
             acc_ref, xn_ref, xnb_ref):
  i = pl.program_id(0)

  @pl.when(i == 0)
  def _init():
    x = x_ref[...]
    nrm = jnp.sqrt(jnp.sum(x * x, axis=1, keepdims=True))
    xn = x / (nrm + 1e-12)
    xn_ref[...] = xn
    xnb_ref[...] = (xn * EXP2_SCALE).astype(jnp.bfloat16)
    acc_ref[...] = jnp.zeros_like(acc_ref)

  nb = neg_ref[...].astype(jnp.bfloat16)
  s = lax.dot_general(xnb_ref[...], nb, (((1,), (1,)), ((), ())),
                      preferred_element_type=jnp.float32)
  e = jnp.exp2(s).astype(jnp.bfloat16)
  ones = jnp.ones((ROWS_PER_BLOCK, 128), dtype=jnp.bfloat16)
  acc_ref[...] += lax.dot_general(e, ones, (((1,), (0,)), ((), ())),
                                  preferred_element_type=jnp.float32)

  @pl.when(i == NUM_BLOCKS - 1)
  def _fini():
    xn = xn_ref[...]
    l_pos = jnp.sum(xn * cp_ref[...], axis=1, keepdims=True)
    t_neg = jnp.sum(xn * cn_ref[...], axis=1, keepdims=True)
    l_neg = acc_ref[:, 0:1] - jnp.exp(t_neg * INV_TEMP) + jnp.exp(l_pos * INV_TEMP)
    per_sample = l_pos * INV_TEMP - jnp.log(l_neg)
    out_ref[...] = jnp.reshape(-jnp.mean(per_sample), (1, 1))


def _tc_loss(x, neg, cp, cn):
  return pl.pallas_call(
      _tc_body,
      grid=(NUM_BLOCKS,),
      in_specs=[
          pl.BlockSpec((B, D), lambda i: (0, 0)),
          pl.BlockSpec((ROWS_PER_BLOCK, D), lambda i: (i, 0)),
          pl.BlockSpec((B, D), lambda i: (0, 0)),
          pl.BlockSpec((B, D), lambda i: (0, 0)),
      ],
      out_specs=pl.BlockSpec((1, 1), lambda i: (0, 0)),
      out_shape=jax.ShapeDtypeStruct((1, 1), jnp.float32),
      scratch_shapes=[
          pltpu.VMEM((B, 128), jnp.float32),
          pltpu.VMEM((B, D), jnp.float32),
          pltpu.VMEM((B, D), jnp.bfloat16),
      ],
  )(x, neg, cp, cn)


def kernel(inputs, pos_protomemory, neg_protomemory, targets, indexes):
  del indexes
  cp, cn = _sc_gather(pos_protomemory, neg_protomemory,
                      targets.astype(jnp.int32))
  loss = _tc_loss(inputs, neg_protomemory, cp, cn)
  return loss[0, 0]


# speedup vs baseline: 1.5582x; 1.5582x over previous
import functools

import jax
import jax.numpy as jnp
from jax import lax
from jax.experimental import pallas as pl
from jax.experimental.pallas import tpu as pltpu
from jax.experimental.pallas import tpu_sc as plsc

B = 1024
D = 256
N = 100000
TEMP = 0.05
INV_TEMP = 1.0 / TEMP

ROWS_PER_BLOCK = 4000
NUM_BLOCKS = N // ROWS_PER_BLOCK

LOG2E = 1.4426950408889634
EXP2_SCALE = INV_TEMP * LOG2E


def _make_sc_gather():
  info = plsc.get_sparse_core_info()
  nw = info.num_cores * info.num_subcores
  b_per_w = B // nw
  mesh = plsc.VectorSubcoreMesh(core_axis_name="c", subcore_axis_name="s")

  @functools.partial(
      pl.kernel,
      mesh=mesh,
      out_type=[
          jax.ShapeDtypeStruct((B, D), jnp.float32),
          jax.ShapeDtypeStruct((B, D), jnp.float32),
      ],
      scratch_types=[
          pltpu.VMEM((b_per_w,), jnp.int32),
          pltpu.VMEM((b_per_w, D), jnp.float32),
          pltpu.VMEM((b_per_w, D), jnp.float32),
          pltpu.SemaphoreType.DMA,
          pltpu.SemaphoreType.DMA,
      ],
  )
  def sc_gather(pos_hbm, neg_hbm, tgt_hbm, out_p, out_n,
                idx_v, rows_p, rows_n, sem_p, sem_n):
    wid = lax.axis_index("s") * info.num_cores + lax.axis_index("c")
    base = wid * b_per_w
    pltpu.sync_copy(tgt_hbm.at[pl.ds(base, b_per_w)], idx_v)
    dma_p = pltpu.async_copy(pos_hbm.at[idx_v], rows_p, sem_p)
    dma_n = pltpu.async_copy(neg_hbm.at[idx_v], rows_n, sem_n)
    dma_p.wait()
    dma_n.wait()
    pltpu.sync_copy(rows_p, out_p.at[pl.ds(base, b_per_w)])
    pltpu.sync_copy(rows_n, out_n.at[pl.ds(base, b_per_w)])

  return sc_gather


_sc_gather_cache = []


def _sc_gather(pos, neg, tgt):
  if not _sc_gather_cache:
    _sc_gather_cache.append(_make_sc_gather())
  return _sc_gather_cache[0](pos, neg, tgt)


def _tc_body(x_ref, neg_ref, cp_ref, cn_ref, out_ref,
             acc_ref, xn_ref, xnb_ref):
  i = pl.program_id(0)

  @pl.when(i == 0)
  def _init():
    x = x_ref[...]
    nrm = jnp.sqrt(jnp.sum(x * x, axis=1, keepdims=True))
    xn = x / (nrm + 1e-12)
    xn_ref[...] = xn
    xnb_ref[...] = (xn * EXP2_SCALE).astype(jnp.bfloat16)
    acc_ref[...] = jnp.zeros_like(acc_ref)

  nb = neg_ref[...].astype(jnp.bfloat16)
  s = lax.dot_general(xnb_ref[...], nb, (((1,), (1,)), ((), ())),
                      preferred_element_type=jnp.float32)
  acc_ref[...] += jnp.sum(jnp.exp2(s), axis=1, keepdims=True)

  @pl.when(i == NUM_BLOCKS - 1)
  def _fini():
    xn = xn_ref[...]
    l_pos = jnp.sum(xn * cp_ref[...], axis=1, keepdims=True)
    t_neg = jnp.sum(xn * cn_ref[...], axis=1, keepdims=True)
    l_neg = acc_ref[...] - jnp.exp(t_neg * INV_TEMP) + jnp.exp(l_pos * INV_TEMP)
    per_sample = l_pos * INV_TEMP - jnp.log(l_neg)
    out_ref[...] = jnp.reshape(-jnp.mean(per_sample), (1, 1))


def _tc_loss(x, neg, cp, cn):
  return pl.pallas_call(
      _tc_body,
      grid=(NUM_BLOCKS,),
      in_specs=[
          pl.BlockSpec((B, D), lambda i: (0, 0)),
          pl.BlockSpec((ROWS_PER_BLOCK, D), lambda i: (i, 0)),
          pl.BlockSpec((B, D), lambda i: (0, 0)),
          pl.BlockSpec((B, D), lambda i: (0, 0)),
      ],
      out_specs=pl.BlockSpec((1, 1), lambda i: (0, 0)),
      out_shape=jax.ShapeDtypeStruct((1, 1), jnp.float32),
      scratch_shapes=[
          pltpu.VMEM((B, 1), jnp.float32),
          pltpu.VMEM((B, D), jnp.float32),
          pltpu.VMEM((B, D), jnp.bfloat16),
      ],
  )(x, neg, cp, cn)


def kernel(inputs, pos_protomemory, neg_protomemory, targets, indexes):
  del indexes
  cp, cn = _sc_gather(pos_protomemory, neg_protomemory,
                      targets.astype(jnp.int32))
  loss = _tc_loss(inputs, neg_protomemory, cp, cn)
  return loss[0, 0]


# speedup vs baseline: 1.6297x; 1.0459x over previous
import functools

import jax
import jax.numpy as jnp
from jax import lax
from jax.experimental import pallas as pl
from jax.experimental.pallas import tpu as pltpu
from jax.experimental.pallas import tpu_sc as plsc

B = 1024
D = 256
N = 100000
TEMP = 0.05
INV_TEMP = 1.0 / TEMP

ROWS_PER_BLOCK = 4000
NUM_BLOCKS = N // ROWS_PER_BLOCK

LOG2E = 1.4426950408889634
EXP2_SCALE = INV_TEMP * LOG2E


def _make_sc_gather():
  info = plsc.get_sparse_core_info()
  nw = info.num_cores * info.num_subcores
  b_per_w = B // nw
  mesh = plsc.VectorSubcoreMesh(core_axis_name="c", subcore_axis_name="s")

  @functools.partial(
      pl.kernel,
      mesh=mesh,
      out_type=[
          jax.ShapeDtypeStruct((B, D), jnp.float32),
          jax.ShapeDtypeStruct((B, D), jnp.float32),
      ],
      scratch_types=[
          pltpu.VMEM((b_per_w,), jnp.int32),
          pltpu.VMEM((b_per_w, D), jnp.float32),
          pltpu.VMEM((b_per_w, D), jnp.float32),
          pltpu.SemaphoreType.DMA,
          pltpu.SemaphoreType.DMA,
      ],
  )
  def sc_gather(pos_hbm, neg_hbm, tgt_hbm, out_p, out_n,
                idx_v, rows_p, rows_n, sem_p, sem_n):
    wid = lax.axis_index("s") * info.num_cores + lax.axis_index("c")
    base = wid * b_per_w
    pltpu.sync_copy(tgt_hbm.at[pl.ds(base, b_per_w)], idx_v)
    dma_p = pltpu.async_copy(pos_hbm.at[idx_v], rows_p, sem_p)
    dma_n = pltpu.async_copy(neg_hbm.at[idx_v], rows_n, sem_n)
    dma_p.wait()
    dma_n.wait()
    pltpu.sync_copy(rows_p, out_p.at[pl.ds(base, b_per_w)])
    pltpu.sync_copy(rows_n, out_n.at[pl.ds(base, b_per_w)])

  return sc_gather


_sc_gather_cache = []


def _sc_gather(pos, neg, tgt):
  if not _sc_gather_cache:
    _sc_gather_cache.append(_make_sc_gather())
  return _sc_gather_cache[0](pos, neg, tgt)


def _stream_body(x_ref, neg_ref, acc_ref, xnb_ref):
  i = pl.program_id(0)

  @pl.when(i == 0)
  def _init():
    x = x_ref[...]
    nrm = jnp.sqrt(jnp.sum(x * x, axis=1, keepdims=True))
    xnb_ref[...] = (x * (EXP2_SCALE / (nrm + 1e-12))).astype(jnp.bfloat16)
    acc_ref[...] = jnp.zeros_like(acc_ref)

  nb = neg_ref[...].astype(jnp.bfloat16)
  s = lax.dot_general(xnb_ref[...], nb, (((1,), (1,)), ((), ())),
                      preferred_element_type=jnp.float32)
  acc_ref[...] += jnp.sum(jnp.exp2(s), axis=1, keepdims=True)


def _exp_sums(x, neg):
  return pl.pallas_call(
      _stream_body,
      grid=(NUM_BLOCKS,),
      in_specs=[
          pl.BlockSpec((B, D), lambda i: (0, 0)),
          pl.BlockSpec((ROWS_PER_BLOCK, D), lambda i: (i, 0)),
      ],
      out_specs=pl.BlockSpec((B, 1), lambda i: (0, 0)),
      out_shape=jax.ShapeDtypeStruct((B, 1), jnp.float32),
      scratch_shapes=[
          pltpu.VMEM((B, D), jnp.bfloat16),
      ],
  )(x, neg)


def _finish_body(x_ref, acc_ref, cp_ref, cn_ref, out_ref):
  x = x_ref[...]
  nrm = jnp.sqrt(jnp.sum(x * x, axis=1, keepdims=True))
  xn = x / (nrm + 1e-12)
  l_pos = jnp.sum(xn * cp_ref[...], axis=1, keepdims=True)
  t_neg = jnp.sum(xn * cn_ref[...], axis=1, keepdims=True)
  l_neg = acc_ref[...] - jnp.exp(t_neg * INV_TEMP) + jnp.exp(l_pos * INV_TEMP)
  per_sample = l_pos * INV_TEMP - jnp.log(l_neg)
  out_ref[...] = jnp.reshape(-jnp.mean(per_sample), (1, 1))


def _finish(x, acc, cp, cn):
  return pl.pallas_call(
      _finish_body,
      out_shape=jax.ShapeDtypeStruct((1, 1), jnp.float32),
  )(x, acc, cp, cn)


def kernel(inputs, pos_protomemory, neg_protomemory, targets, indexes):
  del indexes
  cp, cn = _sc_gather(pos_protomemory, neg_protomemory,
                      targets.astype(jnp.int32))
  acc = _exp_sums(inputs, neg_protomemory)
  loss = _finish(inputs, acc, cp, cn)
  return loss[0, 0]


# speedup vs baseline: 1.7101x; 1.0493x over previous
import functools

import jax
import jax.numpy as jnp
from jax import lax
from jax.experimental import pallas as pl
from jax.experimental.pallas import tpu as pltpu
from jax.experimental.pallas import tpu_sc as plsc

B = 1024
D = 256
N = 100000
TEMP = 0.05
INV_TEMP = 1.0 / TEMP

ROWS_PER_BLOCK = 4000
NUM_BLOCKS = N // ROWS_PER_BLOCK

LOG2E = 1.4426950408889634
EXP2_SCALE = INV_TEMP * LOG2E


def _make_sc_gather():
  info = plsc.get_sparse_core_info()
  nw = info.num_cores * info.num_subcores
  b_per_w = B // nw
  mesh = plsc.VectorSubcoreMesh(core_axis_name="c", subcore_axis_name="s")

  @functools.partial(
      pl.kernel,
      mesh=mesh,
      out_type=[
          jax.ShapeDtypeStruct((B, D), jnp.float32),
          jax.ShapeDtypeStruct((B, D), jnp.float32),
      ],
      scratch_types=[
          pltpu.VMEM((b_per_w,), jnp.int32),
          pltpu.VMEM((b_per_w, D), jnp.float32),
          pltpu.VMEM((b_per_w, D), jnp.float32),
          pltpu.SemaphoreType.DMA,
          pltpu.SemaphoreType.DMA,
      ],
  )
  def sc_gather(pos_hbm, neg_hbm, tgt_hbm, out_p, out_n,
                idx_v, rows_p, rows_n, sem_p, sem_n):
    wid = lax.axis_index("s") * info.num_cores + lax.axis_index("c")
    base = wid * b_per_w
    pltpu.sync_copy(tgt_hbm.at[pl.ds(base, b_per_w)], idx_v)
    dma_p = pltpu.async_copy(pos_hbm.at[idx_v], rows_p, sem_p)
    dma_n = pltpu.async_copy(neg_hbm.at[idx_v], rows_n, sem_n)
    dma_p.wait()
    dma_n.wait()
    pltpu.sync_copy(rows_p, out_p.at[pl.ds(base, b_per_w)])
    pltpu.sync_copy(rows_n, out_n.at[pl.ds(base, b_per_w)])

  return sc_gather


_sc_gather_cache = []


def _sc_gather(pos, neg, tgt):
  if not _sc_gather_cache:
    _sc_gather_cache.append(_make_sc_gather())
  return _sc_gather_cache[0](pos, neg, tgt)


def _stream_body(x_ref, neg_ref, acc_ref, xnb_ref):
  i = pl.program_id(0)

  @pl.when(i == 0)
  def _init():
    x = x_ref[...]
    nrm = jnp.sqrt(jnp.sum(x * x, axis=1, keepdims=True))
    xnb_ref[...] = (x * (EXP2_SCALE / (nrm + 1e-12))).astype(jnp.bfloat16)
    acc_ref[...] = jnp.zeros_like(acc_ref)

  nb = neg_ref[...].astype(jnp.bfloat16)
  s = lax.dot_general(nb, xnb_ref[...], (((1,), (1,)), ((), ())),
                      preferred_element_type=jnp.float32)
  e = jnp.exp2(s).reshape(ROWS_PER_BLOCK // 8, 8, B)
  acc_ref[...] += jnp.sum(e, axis=0)


def _exp_sums(x, neg):
  return pl.pallas_call(
      _stream_body,
      grid=(NUM_BLOCKS,),
      in_specs=[
          pl.BlockSpec((B, D), lambda i: (0, 0)),
          pl.BlockSpec((ROWS_PER_BLOCK, D), lambda i: (i, 0)),
      ],
      out_specs=pl.BlockSpec((8, B), lambda i: (0, 0)),
      out_shape=jax.ShapeDtypeStruct((8, B), jnp.float32),
      scratch_shapes=[
          pltpu.VMEM((B, D), jnp.bfloat16),
      ],
  )(x, neg)


def _finish_body(x_ref, acc_ref, cp_ref, cn_ref, out_ref):
  x = x_ref[...]
  nrm = jnp.sqrt(jnp.sum(x * x, axis=1, keepdims=True))
  xn = x / (nrm + 1e-12)
  l_pos = jnp.sum(xn * cp_ref[...], axis=1, keepdims=True)
  t_neg = jnp.sum(xn * cn_ref[...], axis=1, keepdims=True)
  s_lane = jnp.sum(acc_ref[...], axis=0, keepdims=True)
  row = lax.broadcasted_iota(jnp.int32, (B, B), 0)
  col = lax.broadcasted_iota(jnp.int32, (B, B), 1)
  ident = jnp.where(row == col, 1.0, 0.0).astype(jnp.float32)
  s_row = lax.dot_general(ident, s_lane, (((1,), (1,)), ((), ())),
                          preferred_element_type=jnp.float32)
  l_neg = s_row - jnp.exp(t_neg * INV_TEMP) + jnp.exp(l_pos * INV_TEMP)
  per_sample = l_pos * INV_TEMP - jnp.log(l_neg)
  out_ref[...] = jnp.reshape(-jnp.mean(per_sample), (1, 1))


def _finish(x, acc, cp, cn):
  return pl.pallas_call(
      _finish_body,
      out_shape=jax.ShapeDtypeStruct((1, 1), jnp.float32),
  )(x, acc, cp, cn)


def kernel(inputs, pos_protomemory, neg_protomemory, targets, indexes):
  del indexes
  cp, cn = _sc_gather(pos_protomemory, neg_protomemory,
                      targets.astype(jnp.int32))
  acc = _exp_sums(inputs, neg_protomemory)
  loss = _finish(inputs, acc, cp, cn)
  return loss[0, 0]


# speedup vs baseline: 1.7787x; 1.0401x over previous
import functools

import jax
import jax.numpy as jnp
from jax import lax
from jax.experimental import pallas as pl
from jax.experimental.pallas import tpu as pltpu
from jax.experimental.pallas import tpu_sc as plsc

B = 1024
D = 256
N = 100000
TEMP = 0.05
INV_TEMP = 1.0 / TEMP

ROWS_PER_BLOCK = 5000
NUM_BLOCKS = N // ROWS_PER_BLOCK

LOG2E = 1.4426950408889634
EXP2_SCALE = INV_TEMP * LOG2E


def _make_sc_dots():
  info = plsc.get_sparse_core_info()
  nw = info.num_cores * info.num_subcores
  b_per_w = B // nw
  nl = info.num_lanes
  nchunk = D // nl
  mesh = plsc.VectorSubcoreMesh(core_axis_name="c", subcore_axis_name="s")

  @functools.partial(
      pl.kernel,
      mesh=mesh,
      out_type=[
          jax.ShapeDtypeStruct((B,), jnp.float32),
          jax.ShapeDtypeStruct((B,), jnp.float32),
          jax.ShapeDtypeStruct((B,), jnp.float32),
      ],
      scratch_types=[
          pltpu.VMEM((b_per_w,), jnp.int32),
          pltpu.VMEM((b_per_w, D), jnp.float32),
          pltpu.VMEM((b_per_w, D), jnp.float32),
          pltpu.VMEM((b_per_w, D), jnp.float32),
          pltpu.VMEM((b_per_w,), jnp.float32),
          pltpu.VMEM((b_per_w,), jnp.float32),
          pltpu.VMEM((b_per_w,), jnp.float32),
          pltpu.SemaphoreType.DMA,
          pltpu.SemaphoreType.DMA,
      ],
  )
  def sc_dots(x_hbm, pos_hbm, neg_hbm, tgt_hbm, out_up, out_un, out_q,
              idx_v, x_v, rows_p, rows_n, up_v, un_v, q_v, sem_p, sem_n):
    wid = lax.axis_index("s") * info.num_cores + lax.axis_index("c")
    base = wid * b_per_w
    pltpu.sync_copy(tgt_hbm.at[pl.ds(base, b_per_w)], idx_v)
    dma_p = pltpu.async_copy(pos_hbm.at[idx_v], rows_p, sem_p)
    dma_n = pltpu.async_copy(neg_hbm.at[idx_v], rows_n, sem_n)
    pltpu.sync_copy(x_hbm.at[pl.ds(base, b_per_w)], x_v)
    dma_p.wait()
    dma_n.wait()

    z = jnp.zeros((nl,), jnp.float32)
    lane = lax.iota(jnp.int32, nl)

    dnums = lax.GatherDimensionNumbers(offset_dims=(), collapsed_slice_dims=(0,),
                                       start_index_map=(0,))

    def lane_total(v):
      for sh in (8, 4, 2, 1):
        perm = lane ^ sh
        shuf = lax.gather(v, perm[:, None], dnums, slice_sizes=(1,),
                          mode=lax.GatherScatterMode.PROMISE_IN_BOUNDS)
        v = v + shuf
      return v

    for g in range(b_per_w // nl):
      def row16_body(j, res):
        res_up, res_un, res_q = res
        r = g * nl + j

        def chunk_body(c, carry):
          a_up, a_un, a_q = carry
          xv = x_v[r, pl.ds(c * nl, nl)]
          return (a_up + xv * rows_p[r, pl.ds(c * nl, nl)],
                  a_un + xv * rows_n[r, pl.ds(c * nl, nl)],
                  a_q + xv * xv)

        a_up, a_un, a_q = lax.fori_loop(0, nchunk, chunk_body, (z, z, z))
        sel = lane == j
        return (jnp.where(sel, lane_total(a_up), res_up),
                jnp.where(sel, lane_total(a_un), res_un),
                jnp.where(sel, lane_total(a_q), res_q))

      res_up, res_un, res_q = lax.fori_loop(0, nl, row16_body, (z, z, z))
      up_v[pl.ds(g * nl, nl)] = res_up
      un_v[pl.ds(g * nl, nl)] = res_un
      q_v[pl.ds(g * nl, nl)] = res_q
    pltpu.sync_copy(up_v, out_up.at[pl.ds(base, b_per_w)])
    pltpu.sync_copy(un_v, out_un.at[pl.ds(base, b_per_w)])
    pltpu.sync_copy(q_v, out_q.at[pl.ds(base, b_per_w)])

  return sc_dots


_sc_dots_cache = []


def _sc_dots(x, pos, neg, tgt):
  if not _sc_dots_cache:
    _sc_dots_cache.append(_make_sc_dots())
  return _sc_dots_cache[0](x, pos, neg, tgt)


def _stream_body(x_ref, neg_ref, acc_ref, xnb_ref):
  i = pl.program_id(0)

  @pl.when(i == 0)
  def _init():
    x = x_ref[...]
    nrm = jnp.sqrt(jnp.sum(x * x, axis=1, keepdims=True))
    xnb_ref[...] = (x * (EXP2_SCALE / (nrm + 1e-12))).astype(jnp.bfloat16)
    acc_ref[...] = jnp.zeros_like(acc_ref)

  nb = neg_ref[...].astype(jnp.bfloat16)
  s = lax.dot_general(nb, xnb_ref[...], (((1,), (1,)), ((), ())),
                      preferred_element_type=jnp.float32)
  e = jnp.exp2(s).reshape(ROWS_PER_BLOCK // 8, 8, B)
  acc_ref[...] += jnp.sum(e, axis=0)


def _exp_sums(x, neg):
  return pl.pallas_call(
      _stream_body,
      grid=(NUM_BLOCKS,),
      in_specs=[
          pl.BlockSpec((B, D), lambda i: (0, 0)),
          pl.BlockSpec((ROWS_PER_BLOCK, D), lambda i: (i, 0)),
      ],
      out_specs=pl.BlockSpec((8, B), lambda i: (0, 0)),
      out_shape=jax.ShapeDtypeStruct((8, B), jnp.float32),
      scratch_shapes=[
          pltpu.VMEM((B, D), jnp.bfloat16),
      ],
  )(x, neg)


def _finish_body(acc_ref, up_ref, un_ref, q_ref, out_ref):
  s_lane = jnp.sum(acc_ref[...], axis=0, keepdims=True)
  inv_nrm = 1.0 / (jnp.sqrt(q_ref[...]) + 1e-12)
  l_pos = up_ref[...] * inv_nrm
  t_neg = un_ref[...] * inv_nrm
  l_neg = s_lane - jnp.exp(t_neg * INV_TEMP) + jnp.exp(l_pos * INV_TEMP)
  per_sample = l_pos * INV_TEMP - jnp.log(l_neg)
  out_ref[...] = jnp.reshape(-jnp.mean(per_sample), (1, 1))


def _finish(acc, up, un, q):
  return pl.pallas_call(
      _finish_body,
      out_shape=jax.ShapeDtypeStruct((1, 1), jnp.float32),
  )(acc, up, un, q)


def kernel(inputs, pos_protomemory, neg_protomemory, targets, indexes):
  del indexes
  up, un, q = _sc_dots(inputs, pos_protomemory, neg_protomemory,
                       targets.astype(jnp.int32))
  acc = _exp_sums(inputs, neg_protomemory)
  loss = _finish(acc, up.reshape(1, B), un.reshape(1, B), q.reshape(1, B))
  return loss[0, 0]
